# baseline (speedup 1.0000x reference)
import functools

import jax
import jax.numpy as jnp
from jax import lax
from jax.experimental import pallas as pl
from jax.experimental.pallas import tpu as pltpu
from jax.experimental.pallas import tpu_sc as plsc

DIM = 1024
FF = 4096
E = 8
K = 2
N = 4096
NK = N * K
NC, NS, L = 2, 16, 16
NW = NC * NS
S = NK // NW
TPT = N // NW
BLK = 256
P = NK + E * BLK
RB = P // BLK
FFC = 512
NFF = FF // FFC
RN = 1024
CH = 64
CH2 = 32

@functools.cache
def _mesh():
    return plsc.VectorSubcoreMesh(
        core_axis_name="c", subcore_axis_name="s",
        num_cores=NC, num_subcores=NS)


def _gather16(src, idx):
    return lax.gather(
        src,
        idx[:, None],
        lax.GatherDimensionNumbers(
            offset_dims=(), collapsed_slice_dims=(0,), start_index_map=(0,)),
        (1,),
        mode=lax.GatherScatterMode.PROMISE_IN_BOUNDS,
    )


def _router_body(x_ref, wr_ref, a1_ref, a2_ref, w1_ref, w2_ref, c1_ref, c2_ref):
    xb = x_ref[...]
    logits = jax.lax.dot_general(
        xb, wr_ref[...], (((1,), (1,)), ((), ())),
        preferred_element_type=jnp.float32)
    ids = lax.broadcasted_iota(jnp.int32, (RN, E), 1)
    m1 = jnp.max(logits, axis=1, keepdims=True)
    a1 = jnp.min(jnp.where(logits == m1, ids, E), axis=1)
    neg = jnp.finfo(jnp.float32).min
    l2 = jnp.where(ids == a1[:, None], neg, logits)
    m2 = jnp.max(l2, axis=1, keepdims=True)
    a2 = jnp.min(jnp.where(l2 == m2, ids, E), axis=1)
    g = 1.0 / (1.0 + jnp.exp(m2[:, 0] - m1[:, 0]))
    a1_ref[...] = a1
    a2_ref[...] = a2
    w1_ref[...] = g
    w2_ref[...] = 1.0 - g
    ids16 = lax.broadcasted_iota(jnp.int32, (RN, L), 1)
    grp = (lax.broadcasted_iota(jnp.int32, (RN // S, RN), 1) // S ==
           lax.broadcasted_iota(jnp.int32, (RN // S, RN), 0)).astype(jnp.float32)
    oh1 = (ids16 == a1[:, None]).astype(jnp.float32)
    oh2 = (ids16 == a2[:, None]).astype(jnp.float32)
    c1_ref[...] = jnp.dot(grp, oh1, preferred_element_type=jnp.float32
                          ).astype(jnp.int32).reshape(1, RN // S, L)
    c2_ref[...] = jnp.dot(grp, oh2, preferred_element_type=jnp.float32
                          ).astype(jnp.int32).reshape(1, RN // S, L)


def _router(x_flat, Wr):
    nblk = N // RN
    return pl.pallas_call(
        _router_body,
        grid=(nblk,),
        in_specs=[
            pl.BlockSpec((RN, DIM), lambda b: (b, 0)),
            pl.BlockSpec((E, DIM), lambda b: (0, 0)),
        ],
        out_specs=[
            pl.BlockSpec((RN,), lambda b: (b,)),
            pl.BlockSpec((RN,), lambda b: (b,)),
            pl.BlockSpec((RN,), lambda b: (b,)),
            pl.BlockSpec((RN,), lambda b: (b,)),
            pl.BlockSpec((1, RN // S, L), lambda b: (b, 0, 0)),
            pl.BlockSpec((1, RN // S, L), lambda b: (b, 0, 0)),
        ],
        out_shape=[
            jax.ShapeDtypeStruct((N,), jnp.int32),
            jax.ShapeDtypeStruct((N,), jnp.int32),
            jax.ShapeDtypeStruct((N,), jnp.float32),
            jax.ShapeDtypeStruct((N,), jnp.float32),
            jax.ShapeDtypeStruct((nblk, RN // S, L), jnp.int32),
            jax.ShapeDtypeStruct((nblk, RN // S, L), jnp.int32),
        ],
    )(x_flat, Wr)


@functools.cache
def _dispatch_fn():
    return functools.partial(
        pl.kernel,
        out_type=[
            jax.ShapeDtypeStruct((NK,), jnp.int32),
            jax.ShapeDtypeStruct((P, DIM), jnp.float32),
            jax.ShapeDtypeStruct((P, 128), jnp.float32),
            jax.ShapeDtypeStruct((RB,), jnp.int32),
        ],
        mesh=_mesh(),
        scratch_types=[
            pltpu.VMEM((NW, L), jnp.int32),
            pltpu.VMEM((S,), jnp.int32),
            pltpu.VMEM((S // CH, CH), jnp.int32),
            pltpu.VMEM((CH, DIM), jnp.float32),
            pltpu.VMEM((CH, 128), jnp.float32),
            pltpu.VMEM((3 * L,), jnp.int32),
            pltpu.SemaphoreType.DMA,
            pltpu.SemaphoreType.DMA,
        ],
        compiler_params=pltpu.CompilerParams(needs_layout_passes=False),
    )(_dispatch_body)


def _dispatch_body(e_hbm, wq_hbm, cnt_hbm, x_hbm,
              dest_hbm, xp_hbm, sw_hbm, beo_hbm,
              cnt_v, ev, destv, xr, wr, beov, sem1, sem2):
    wid = lax.axis_index("s") * NC + lax.axis_index("c")
    pltpu.sync_copy(cnt_hbm, cnt_v)
    pltpu.sync_copy(e_hbm.at[pl.ds(wid * S, S)], ev)

    lane = lax.broadcasted_iota(jnp.int32, (L,), 0)
    zero = jnp.zeros((L,), jnp.int32)
    tot = zero
    pre = zero
    for j in range(NW):
        row = cnt_v[j]
        tot = tot + row
        pre = pre + jnp.where(jnp.full((L,), j, jnp.int32) < wid, row, zero)
    padded = ((tot + (BLK - 1)) >> 8) << 8
    incl = plsc.cumsum(padded)
    po = incl - padded
    base = po + pre

    def dbody(j, run):
        v = ev[pl.ds(j * L, L)]
        rank = zero
        hist = zero
        for e in range(E):
            m = v == e
            mi = m.astype(jnp.int32)
            inc = plsc.cumsum(mi)
            rank = jnp.where(m, inc - 1, rank)
            hist = jnp.where(lane == e, jnp.sum(mi), hist)
        dvec = _gather16(base + run, v) + rank
        destv[j >> 2, pl.ds((j & 3) * L, L)] = dvec
        return run + hist

    lax.fori_loop(0, S // L, dbody, zero)

    tb = jnp.where(wid < NW // 2, wid * S, wid * S - N)
    for c in range(S // CH):
        pltpu.sync_copy(destv.at[c],
                        dest_hbm.at[pl.ds(wid * S + c * CH, CH)])
        pltpu.sync_copy(x_hbm.at[pl.ds(tb + c * CH, CH)], xr)
        pltpu.sync_copy(wq_hbm.at[pl.ds(wid * S + c * CH, CH)], wr)
        cp1 = pltpu.async_copy(xr, xp_hbm.at[destv.at[c]], sem1)
        cp2 = pltpu.async_copy(wr, sw_hbm.at[destv.at[c]], sem2)
        cp1.wait()
        cp2.wait()

    @pl.when(wid == 0)
    def _():
        for jb in range(3):
            bstart = (lane + jb * L) * BLK
            acc = zero
            for e in range(E):
                th = _gather16(po, jnp.full((L,), e, jnp.int32))
                acc = acc + jnp.where(th <= bstart, 1, 0)
            beov[pl.ds(jb * L, L)] = acc - 1
        pltpu.sync_copy(beov.at[pl.ds(0, RB)], beo_hbm)


def _ffn_body(be_ref, xp_ref, sw_ref, w1_ref, w2_ref, y_ref, acc):
    fc = pl.program_id(1)
    h = jnp.dot(xp_ref[...].astype(jnp.bfloat16), w1_ref[0],
                preferred_element_type=jnp.float32)
    h = 0.5 * h * (1.0 + lax.erf(h * (2.0 ** -0.5)))
    p = jnp.dot(h.astype(jnp.bfloat16), w2_ref[0],
                preferred_element_type=jnp.float32)

    @pl.when(fc == 0)
    def _():
        acc[...] = p

    @pl.when(fc > 0)
    def _():
        acc[...] += p

    @pl.when(fc == NFF - 1)
    def _():
        y_ref[...] = acc[...] * sw_ref[...][:, :1]


def _ffn(beo, xp, sw, W1, W2):
    grid_spec = pltpu.PrefetchScalarGridSpec(
        num_scalar_prefetch=1,
        grid=(RB, NFF),
        in_specs=[
            pl.BlockSpec((BLK, DIM), lambda rb, fc, be: (rb, 0)),
            pl.BlockSpec((BLK, 128), lambda rb, fc, be: (rb, 0)),
            pl.BlockSpec((1, DIM, FFC), lambda rb, fc, be: (be[rb], 0, fc)),
            pl.BlockSpec((1, FFC, DIM), lambda rb, fc, be: (be[rb], fc, 0)),
        ],
        out_specs=pl.BlockSpec((BLK, DIM), lambda rb, fc, be: (rb, 0)),
        scratch_shapes=[pltpu.VMEM((BLK, DIM), jnp.float32)],
    )
    return pl.pallas_call(
        _ffn_body,
        grid_spec=grid_spec,
        out_shape=jax.ShapeDtypeStruct((P, DIM), jnp.float32),
        compiler_params=pltpu.CompilerParams(
            dimension_semantics=("arbitrary", "arbitrary")),
    )(beo, xp, sw, W1, W2)


@functools.cache
def _combine_fn():
    return functools.partial(
        pl.kernel,
        out_type=jax.ShapeDtypeStruct((N, DIM), jnp.float32),
        mesh=_mesh(),
        scratch_types=[
            pltpu.VMEM((CH2,), jnp.int32),
            pltpu.VMEM((CH2,), jnp.int32),
            pltpu.VMEM((CH2, DIM), jnp.float32),
            pltpu.VMEM((CH2, DIM), jnp.float32),
            pltpu.VMEM((CH2, DIM), jnp.float32),
            pltpu.SemaphoreType.DMA,
            pltpu.SemaphoreType.DMA,
        ],
        compiler_params=pltpu.CompilerParams(needs_layout_passes=False),
    )(_combine_body)


def _combine_body(y_hbm, dest_hbm, out_hbm, d0, d1, r0, r1, ob, sem1, sem2):
    wid = lax.axis_index("s") * NC + lax.axis_index("c")
    t0 = wid * TPT
    for c in range(TPT // CH2):
        tbase = t0 + c * CH2
        pltpu.sync_copy(dest_hbm.at[pl.ds(tbase, CH2)], d0)
        pltpu.sync_copy(dest_hbm.at[pl.ds(N + tbase, CH2)], d1)
        cp1 = pltpu.async_copy(y_hbm.at[d0], r0, sem1)
        cp2 = pltpu.async_copy(y_hbm.at[d1], r1, sem2)
        cp1.wait()
        cp2.wait()

        def cbody(t, _):
            for kk in range(DIM // L):
                sl = pl.ds(kk * L, L)
                ob[t, sl] = r0[t, sl] + r1[t, sl]
            return 0

        lax.fori_loop(0, CH2, cbody, 0)
        pltpu.sync_copy(ob, out_hbm.at[pl.ds(tbase, CH2)])


def kernel(x, Wr, W1, W2):
    Bb, Tt, D = x.shape
    x_flat = x.reshape(N, D)
    a1, a2, w1v, w2v, c1, c2 = _router(x_flat, Wr)
    eflat = jnp.concatenate([a1, a2])
    wq = jnp.broadcast_to(jnp.concatenate([w1v, w2v])[:, None], (NK, 128))
    cnt = jnp.concatenate([c1.reshape(NW // 2, L), c2.reshape(NW // 2, L)])
    dest, xp, sw, beo = _dispatch_fn()(eflat, wq, cnt, x_flat)
    y = _ffn(beo, xp, sw,
             W1.astype(jnp.bfloat16), W2.astype(jnp.bfloat16))
    out = _combine_fn()(y, dest)
    return out.reshape(Bb, Tt, D)


# speedup vs baseline: 1.1797x; 1.1797x over previous
import functools

import jax
import jax.numpy as jnp
from jax import lax
from jax.experimental import pallas as pl
from jax.experimental.pallas import tpu as pltpu
from jax.experimental.pallas import tpu_sc as plsc

DIM = 1024
FF = 4096
E = 8
K = 2
N = 4096
NK = N * K
NC, NS, L = 2, 16, 16
NW = NC * NS
S = NK // NW
TPT = N // NW
BLK = 512
BSH = 9
P = NK + E * BLK
RB = P // BLK
FFC = 512
NFF = FF // FFC
RN = 1024
CH = 64
CH2 = 32

@functools.cache
def _mesh():
    return plsc.VectorSubcoreMesh(
        core_axis_name="c", subcore_axis_name="s",
        num_cores=NC, num_subcores=NS)


def _gather16(src, idx):
    return lax.gather(
        src,
        idx[:, None],
        lax.GatherDimensionNumbers(
            offset_dims=(), collapsed_slice_dims=(0,), start_index_map=(0,)),
        (1,),
        mode=lax.GatherScatterMode.PROMISE_IN_BOUNDS,
    )


def _router_body(x_ref, wr_ref, a1_ref, a2_ref, w1_ref, w2_ref, c1_ref, c2_ref):
    xb = x_ref[...]
    logits = jax.lax.dot_general(
        xb, wr_ref[...], (((1,), (1,)), ((), ())),
        preferred_element_type=jnp.float32)
    ids = lax.broadcasted_iota(jnp.int32, (RN, E), 1)
    m1 = jnp.max(logits, axis=1, keepdims=True)
    a1 = jnp.min(jnp.where(logits == m1, ids, E), axis=1)
    neg = jnp.finfo(jnp.float32).min
    l2 = jnp.where(ids == a1[:, None], neg, logits)
    m2 = jnp.max(l2, axis=1, keepdims=True)
    a2 = jnp.min(jnp.where(l2 == m2, ids, E), axis=1)
    g = 1.0 / (1.0 + jnp.exp(m2[:, 0] - m1[:, 0]))
    a1_ref[...] = a1
    a2_ref[...] = a2
    w1_ref[...] = g
    w2_ref[...] = 1.0 - g
    ids16 = lax.broadcasted_iota(jnp.int32, (RN, L), 1)
    grp = (lax.broadcasted_iota(jnp.int32, (RN // S, RN), 1) // S ==
           lax.broadcasted_iota(jnp.int32, (RN // S, RN), 0)).astype(jnp.float32)
    oh1 = (ids16 == a1[:, None]).astype(jnp.float32)
    oh2 = (ids16 == a2[:, None]).astype(jnp.float32)
    c1_ref[...] = jnp.dot(grp, oh1, preferred_element_type=jnp.float32
                          ).astype(jnp.int32).reshape(1, RN // S, L)
    c2_ref[...] = jnp.dot(grp, oh2, preferred_element_type=jnp.float32
                          ).astype(jnp.int32).reshape(1, RN // S, L)


def _router(x_flat, Wr):
    nblk = N // RN
    return pl.pallas_call(
        _router_body,
        grid=(nblk,),
        in_specs=[
            pl.BlockSpec((RN, DIM), lambda b: (b, 0)),
            pl.BlockSpec((E, DIM), lambda b: (0, 0)),
        ],
        out_specs=[
            pl.BlockSpec((RN,), lambda b: (b,)),
            pl.BlockSpec((RN,), lambda b: (b,)),
            pl.BlockSpec((RN,), lambda b: (b,)),
            pl.BlockSpec((RN,), lambda b: (b,)),
            pl.BlockSpec((1, RN // S, L), lambda b: (b, 0, 0)),
            pl.BlockSpec((1, RN // S, L), lambda b: (b, 0, 0)),
        ],
        out_shape=[
            jax.ShapeDtypeStruct((N,), jnp.int32),
            jax.ShapeDtypeStruct((N,), jnp.int32),
            jax.ShapeDtypeStruct((N,), jnp.float32),
            jax.ShapeDtypeStruct((N,), jnp.float32),
            jax.ShapeDtypeStruct((nblk, RN // S, L), jnp.int32),
            jax.ShapeDtypeStruct((nblk, RN // S, L), jnp.int32),
        ],
    )(x_flat, Wr)


@functools.cache
def _dispatch_fn():
    return functools.partial(
        pl.kernel,
        out_type=[
            jax.ShapeDtypeStruct((NK,), jnp.int32),
            jax.ShapeDtypeStruct((P, DIM), jnp.float32),
            jax.ShapeDtypeStruct((P, 128), jnp.float32),
            jax.ShapeDtypeStruct((RB,), jnp.int32),
        ],
        mesh=_mesh(),
        scratch_types=[
            pltpu.VMEM((NW, L), jnp.int32),
            pltpu.VMEM((S,), jnp.int32),
            pltpu.VMEM((S // CH, CH), jnp.int32),
            pltpu.VMEM((CH, DIM), jnp.float32),
            pltpu.VMEM((CH, 128), jnp.float32),
            pltpu.VMEM((-(-RB // L) * L,), jnp.int32),
            pltpu.SemaphoreType.DMA,
            pltpu.SemaphoreType.DMA,
        ],
        compiler_params=pltpu.CompilerParams(needs_layout_passes=False),
    )(_dispatch_body)


def _dispatch_body(e_hbm, wq_hbm, cnt_hbm, x_hbm,
              dest_hbm, xp_hbm, sw_hbm, beo_hbm,
              cnt_v, ev, destv, xr, wr, beov, sem1, sem2):
    wid = lax.axis_index("s") * NC + lax.axis_index("c")
    pltpu.sync_copy(cnt_hbm, cnt_v)
    pltpu.sync_copy(e_hbm.at[pl.ds(wid * S, S)], ev)

    lane = lax.broadcasted_iota(jnp.int32, (L,), 0)
    zero = jnp.zeros((L,), jnp.int32)
    tot = zero
    pre = zero
    for j in range(NW):
        row = cnt_v[j]
        tot = tot + row
        pre = pre + jnp.where(jnp.full((L,), j, jnp.int32) < wid, row, zero)
    padded = ((tot + (BLK - 1)) >> BSH) << BSH
    incl = plsc.cumsum(padded)
    po = incl - padded
    base = po + pre

    def dbody(j, run):
        v = ev[pl.ds(j * L, L)]
        rank = zero
        hist = zero
        for e in range(E):
            m = v == e
            mi = m.astype(jnp.int32)
            inc = plsc.cumsum(mi)
            rank = jnp.where(m, inc - 1, rank)
            hist = jnp.where(lane == e, jnp.sum(mi), hist)
        dvec = _gather16(base + run, v) + rank
        destv[j >> 2, pl.ds((j & 3) * L, L)] = dvec
        return run + hist

    lax.fori_loop(0, S // L, dbody, zero)

    tb = jnp.where(wid < NW // 2, wid * S, wid * S - N)
    for c in range(S // CH):
        pltpu.sync_copy(destv.at[c],
                        dest_hbm.at[pl.ds(wid * S + c * CH, CH)])
        pltpu.sync_copy(x_hbm.at[pl.ds(tb + c * CH, CH)], xr)
        pltpu.sync_copy(wq_hbm.at[pl.ds(wid * S + c * CH, CH)], wr)
        cp1 = pltpu.async_copy(xr, xp_hbm.at[destv.at[c]], sem1)
        cp2 = pltpu.async_copy(wr, sw_hbm.at[destv.at[c]], sem2)
        cp1.wait()
        cp2.wait()

    @pl.when(wid == 0)
    def _():
        for jb in range(-(-RB // L)):
            bstart = (lane + jb * L) * BLK
            acc = zero
            for e in range(E):
                th = _gather16(po, jnp.full((L,), e, jnp.int32))
                acc = acc + jnp.where(th <= bstart, 1, 0)
            beov[pl.ds(jb * L, L)] = acc - 1
        pltpu.sync_copy(beov.at[pl.ds(0, RB)], beo_hbm)


def _ffn_body(be_ref, xp_ref, sw_ref, w1_ref, w2_ref, y_ref, acc):
    fc = pl.program_id(1)
    h = jnp.dot(xp_ref[...].astype(jnp.bfloat16), w1_ref[0],
                preferred_element_type=jnp.float32)
    h = 0.5 * h * (1.0 + lax.erf(h * (2.0 ** -0.5)))
    p = jnp.dot(h.astype(jnp.bfloat16), w2_ref[0],
                preferred_element_type=jnp.float32)

    @pl.when(fc == 0)
    def _():
        acc[...] = p

    @pl.when(fc > 0)
    def _():
        acc[...] += p

    @pl.when(fc == NFF - 1)
    def _():
        y_ref[...] = acc[...] * sw_ref[...][:, :1]


def _ffn(beo, xp, sw, W1, W2):
    grid_spec = pltpu.PrefetchScalarGridSpec(
        num_scalar_prefetch=1,
        grid=(RB, NFF),
        in_specs=[
            pl.BlockSpec((BLK, DIM), lambda rb, fc, be: (rb, 0)),
            pl.BlockSpec((BLK, 128), lambda rb, fc, be: (rb, 0)),
            pl.BlockSpec((1, DIM, FFC), lambda rb, fc, be: (be[rb], 0, fc)),
            pl.BlockSpec((1, FFC, DIM), lambda rb, fc, be: (be[rb], fc, 0)),
        ],
        out_specs=pl.BlockSpec((BLK, DIM), lambda rb, fc, be: (rb, 0)),
        scratch_shapes=[pltpu.VMEM((BLK, DIM), jnp.float32)],
    )
    return pl.pallas_call(
        _ffn_body,
        grid_spec=grid_spec,
        out_shape=jax.ShapeDtypeStruct((P, DIM), jnp.float32),
        compiler_params=pltpu.CompilerParams(
            dimension_semantics=("arbitrary", "arbitrary")),
    )(beo, xp, sw, W1, W2)


@functools.cache
def _combine_fn():
    return functools.partial(
        pl.kernel,
        out_type=jax.ShapeDtypeStruct((N, DIM), jnp.float32),
        mesh=_mesh(),
        scratch_types=[
            pltpu.VMEM((CH2,), jnp.int32),
            pltpu.VMEM((CH2,), jnp.int32),
            pltpu.VMEM((CH2, DIM), jnp.float32),
            pltpu.VMEM((CH2, DIM), jnp.float32),
            pltpu.VMEM((CH2, DIM), jnp.float32),
            pltpu.SemaphoreType.DMA,
            pltpu.SemaphoreType.DMA,
        ],
        compiler_params=pltpu.CompilerParams(needs_layout_passes=False),
    )(_combine_body)


def _combine_body(y_hbm, dest_hbm, out_hbm, d0, d1, r0, r1, ob, sem1, sem2):
    wid = lax.axis_index("s") * NC + lax.axis_index("c")
    t0 = wid * TPT
    for c in range(TPT // CH2):
        tbase = t0 + c * CH2
        pltpu.sync_copy(dest_hbm.at[pl.ds(tbase, CH2)], d0)
        pltpu.sync_copy(dest_hbm.at[pl.ds(N + tbase, CH2)], d1)
        cp1 = pltpu.async_copy(y_hbm.at[d0], r0, sem1)
        cp2 = pltpu.async_copy(y_hbm.at[d1], r1, sem2)
        cp1.wait()
        cp2.wait()

        def cbody(t, _):
            for kk in range(DIM // L):
                sl = pl.ds(kk * L, L)
                ob[t, sl] = r0[t, sl] + r1[t, sl]
            return 0

        lax.fori_loop(0, CH2, cbody, 0)
        pltpu.sync_copy(ob, out_hbm.at[pl.ds(tbase, CH2)])


def kernel(x, Wr, W1, W2):
    Bb, Tt, D = x.shape
    x_flat = x.reshape(N, D)
    a1, a2, w1v, w2v, c1, c2 = _router(x_flat, Wr)
    eflat = jnp.concatenate([a1, a2])
    wq = jnp.broadcast_to(jnp.concatenate([w1v, w2v])[:, None], (NK, 128))
    cnt = jnp.concatenate([c1.reshape(NW // 2, L), c2.reshape(NW // 2, L)])
    dest, xp, sw, beo = _dispatch_fn()(eflat, wq, cnt, x_flat)
    y = _ffn(beo, xp, sw,
             W1.astype(jnp.bfloat16), W2.astype(jnp.bfloat16))
    out = _combine_fn()(y, dest)
    return out.reshape(Bb, Tt, D)


# speedup vs baseline: 1.4655x; 1.2423x over previous
import functools

import jax
import jax.numpy as jnp
from jax import lax
from jax.experimental import pallas as pl
from jax.experimental.pallas import tpu as pltpu
from jax.experimental.pallas import tpu_sc as plsc

DIM = 1024
FF = 4096
E = 8
K = 2
N = 4096
NK = N * K
NC, NS, L = 2, 16, 16
NW = NC * NS
S = NK // NW
TPT = N // NW
BLK = 512
BSH = 9
P = NK + E * BLK
RB = P // BLK
FFC = 1024
NFF = FF // FFC
RN = 1024
CH = 64
CH2 = 32

@functools.cache
def _mesh():
    return plsc.VectorSubcoreMesh(
        core_axis_name="c", subcore_axis_name="s",
        num_cores=NC, num_subcores=NS)


def _gather16(src, idx):
    return lax.gather(
        src,
        idx[:, None],
        lax.GatherDimensionNumbers(
            offset_dims=(), collapsed_slice_dims=(0,), start_index_map=(0,)),
        (1,),
        mode=lax.GatherScatterMode.PROMISE_IN_BOUNDS,
    )


def _router_body(x_ref, wr_ref, a1_ref, a2_ref, w1_ref, w2_ref, c1_ref, c2_ref):
    xb = x_ref[...]
    logits = jax.lax.dot_general(
        xb, wr_ref[...], (((1,), (1,)), ((), ())),
        preferred_element_type=jnp.float32)
    ids = lax.broadcasted_iota(jnp.int32, (RN, E), 1)
    m1 = jnp.max(logits, axis=1, keepdims=True)
    a1 = jnp.min(jnp.where(logits == m1, ids, E), axis=1)
    neg = jnp.finfo(jnp.float32).min
    l2 = jnp.where(ids == a1[:, None], neg, logits)
    m2 = jnp.max(l2, axis=1, keepdims=True)
    a2 = jnp.min(jnp.where(l2 == m2, ids, E), axis=1)
    g = 1.0 / (1.0 + jnp.exp(m2[:, 0] - m1[:, 0]))
    a1_ref[...] = a1
    a2_ref[...] = a2
    w1_ref[...] = g
    w2_ref[...] = 1.0 - g
    ids16 = lax.broadcasted_iota(jnp.int32, (RN, L), 1)
    grp = (lax.broadcasted_iota(jnp.int32, (RN // S, RN), 1) // S ==
           lax.broadcasted_iota(jnp.int32, (RN // S, RN), 0)).astype(jnp.float32)
    oh1 = (ids16 == a1[:, None]).astype(jnp.float32)
    oh2 = (ids16 == a2[:, None]).astype(jnp.float32)
    c1_ref[...] = jnp.dot(grp, oh1, preferred_element_type=jnp.float32
                          ).astype(jnp.int32).reshape(1, RN // S, L)
    c2_ref[...] = jnp.dot(grp, oh2, preferred_element_type=jnp.float32
                          ).astype(jnp.int32).reshape(1, RN // S, L)


def _router(x_flat, Wr):
    nblk = N // RN
    return pl.pallas_call(
        _router_body,
        grid=(nblk,),
        in_specs=[
            pl.BlockSpec((RN, DIM), lambda b: (b, 0)),
            pl.BlockSpec((E, DIM), lambda b: (0, 0)),
        ],
        out_specs=[
            pl.BlockSpec((RN,), lambda b: (b,)),
            pl.BlockSpec((RN,), lambda b: (b,)),
            pl.BlockSpec((RN,), lambda b: (b,)),
            pl.BlockSpec((RN,), lambda b: (b,)),
            pl.BlockSpec((1, RN // S, L), lambda b: (b, 0, 0)),
            pl.BlockSpec((1, RN // S, L), lambda b: (b, 0, 0)),
        ],
        out_shape=[
            jax.ShapeDtypeStruct((N,), jnp.int32),
            jax.ShapeDtypeStruct((N,), jnp.int32),
            jax.ShapeDtypeStruct((N,), jnp.float32),
            jax.ShapeDtypeStruct((N,), jnp.float32),
            jax.ShapeDtypeStruct((nblk, RN // S, L), jnp.int32),
            jax.ShapeDtypeStruct((nblk, RN // S, L), jnp.int32),
        ],
    )(x_flat, Wr)


@functools.cache
def _dispatch_fn():
    return functools.partial(
        pl.kernel,
        out_type=[
            jax.ShapeDtypeStruct((NK,), jnp.int32),
            jax.ShapeDtypeStruct((P, DIM), jnp.float32),
            jax.ShapeDtypeStruct((P, 128), jnp.float32),
            jax.ShapeDtypeStruct((RB,), jnp.int32),
            jax.ShapeDtypeStruct((RB,), jnp.int32),
        ],
        mesh=_mesh(),
        scratch_types=[
            pltpu.VMEM((NW, L), jnp.int32),
            pltpu.VMEM((S,), jnp.int32),
            pltpu.VMEM((S // CH, CH), jnp.int32),
            pltpu.VMEM((CH, DIM), jnp.float32),
            pltpu.VMEM((CH, 128), jnp.float32),
            pltpu.VMEM((-(-RB // L) * L,), jnp.int32),
            pltpu.VMEM((-(-RB // L) * L,), jnp.int32),
            pltpu.SemaphoreType.DMA,
            pltpu.SemaphoreType.DMA,
        ],
        compiler_params=pltpu.CompilerParams(needs_layout_passes=False),
    )(_dispatch_body)


def _dispatch_body(e_hbm, wq_hbm, cnt_hbm, x_hbm,
                   dest_hbm, xp_hbm, sw_hbm, beo_hbm, beu_hbm,
                   cnt_v, ev, destv, xr, wr, beov, beuv, sem1, sem2):
    wid = lax.axis_index("s") * NC + lax.axis_index("c")
    pltpu.sync_copy(cnt_hbm, cnt_v)
    pltpu.sync_copy(e_hbm.at[pl.ds(wid * S, S)], ev)

    lane = lax.broadcasted_iota(jnp.int32, (L,), 0)
    zero = jnp.zeros((L,), jnp.int32)
    tot = zero
    pre = zero
    for j in range(NW):
        row = cnt_v[j]
        tot = tot + row
        pre = pre + jnp.where(jnp.full((L,), j, jnp.int32) < wid, row, zero)
    padded = ((tot + (BLK - 1)) >> BSH) << BSH
    incl = plsc.cumsum(padded)
    po = incl - padded
    base = po + pre

    def dbody(j, run):
        v = ev[pl.ds(j * L, L)]
        rank = zero
        hist = zero
        for e in range(E):
            m = v == e
            mi = m.astype(jnp.int32)
            inc = plsc.cumsum(mi)
            rank = jnp.where(m, inc - 1, rank)
            hist = jnp.where(lane == e, jnp.sum(mi), hist)
        dvec = _gather16(base + run, v) + rank
        destv[j >> 2, pl.ds((j & 3) * L, L)] = dvec
        return run + hist

    lax.fori_loop(0, S // L, dbody, zero)

    tb = jnp.where(wid < NW // 2, wid * S, wid * S - N)
    for c in range(S // CH):
        pltpu.sync_copy(destv.at[c],
                        dest_hbm.at[pl.ds(wid * S + c * CH, CH)])
        pltpu.sync_copy(x_hbm.at[pl.ds(tb + c * CH, CH)], xr)
        pltpu.sync_copy(wq_hbm.at[pl.ds(wid * S + c * CH, CH)], wr)
        cp1 = pltpu.async_copy(xr, xp_hbm.at[destv.at[c]], sem1)
        cp2 = pltpu.async_copy(wr, sw_hbm.at[destv.at[c]], sem2)
        cp1.wait()
        cp2.wait()

    @pl.when(wid == 0)
    def _():
        for jb in range(-(-RB // L)):
            bstart = (lane + jb * L) * BLK
            acc = zero
            for e in range(E):
                th = _gather16(po, jnp.full((L,), e, jnp.int32))
                acc = acc + jnp.where(th <= bstart, 1, 0)
            bev = acc - 1
            beov[pl.ds(jb * L, L)] = bev
            rend = _gather16(po + tot, bev)
            beuv[pl.ds(jb * L, L)] = jnp.where(bstart < rend, 1, 0)
        pltpu.sync_copy(beov.at[pl.ds(0, RB)], beo_hbm)
        pltpu.sync_copy(beuv.at[pl.ds(0, RB)], beu_hbm)


def _ffn_body(be_ref, bu_ref, xp_ref, sw_ref, w1_ref, w2_ref, y_ref, acc):
    rb = pl.program_id(0)
    fc = pl.program_id(1)

    @pl.when(bu_ref[rb] == 1)
    def _():
        h = jnp.dot(xp_ref[...].astype(jnp.bfloat16), w1_ref[0],
                    preferred_element_type=jnp.float32)
        h = 0.5 * h * (1.0 + lax.erf(h * (2.0 ** -0.5)))
        p = jnp.dot(h.astype(jnp.bfloat16), w2_ref[0],
                    preferred_element_type=jnp.float32)

        @pl.when(fc == 0)
        def _():
            acc[...] = p

        @pl.when(fc > 0)
        def _():
            acc[...] += p

        @pl.when(fc == NFF - 1)
        def _():
            y_ref[...] = acc[...] * sw_ref[...][:, :1]


def _ffn(beo, beu, xp, sw, W1, W2):
    grid_spec = pltpu.PrefetchScalarGridSpec(
        num_scalar_prefetch=2,
        grid=(RB, NFF),
        in_specs=[
            pl.BlockSpec((BLK, DIM), lambda rb, fc, be, bu: (rb * bu[rb], 0)),
            pl.BlockSpec((BLK, 128), lambda rb, fc, be, bu: (rb * bu[rb], 0)),
            pl.BlockSpec((1, DIM, FFC),
                         lambda rb, fc, be, bu: (be[rb], 0, fc * bu[rb])),
            pl.BlockSpec((1, FFC, DIM),
                         lambda rb, fc, be, bu: (be[rb], fc * bu[rb], 0)),
        ],
        out_specs=pl.BlockSpec((BLK, DIM), lambda rb, fc, be, bu: (rb, 0)),
        scratch_shapes=[pltpu.VMEM((BLK, DIM), jnp.float32)],
    )
    return pl.pallas_call(
        _ffn_body,
        grid_spec=grid_spec,
        out_shape=jax.ShapeDtypeStruct((P, DIM), jnp.float32),
        compiler_params=pltpu.CompilerParams(
            dimension_semantics=("arbitrary", "arbitrary")),
    )(beo, beu, xp, sw, W1, W2)


@functools.cache
def _combine_fn():
    return functools.partial(
        pl.kernel,
        out_type=jax.ShapeDtypeStruct((N, DIM), jnp.float32),
        mesh=_mesh(),
        scratch_types=[
            pltpu.VMEM((CH2,), jnp.int32),
            pltpu.VMEM((CH2,), jnp.int32),
            pltpu.VMEM((CH2, DIM), jnp.float32),
            pltpu.VMEM((CH2, DIM), jnp.float32),
            pltpu.VMEM((CH2, DIM), jnp.float32),
            pltpu.SemaphoreType.DMA,
            pltpu.SemaphoreType.DMA,
        ],
        compiler_params=pltpu.CompilerParams(needs_layout_passes=False),
    )(_combine_body)


def _combine_body(y_hbm, dest_hbm, out_hbm, d0, d1, r0, r1, ob, sem1, sem2):
    wid = lax.axis_index("s") * NC + lax.axis_index("c")
    t0 = wid * TPT
    for c in range(TPT // CH2):
        tbase = t0 + c * CH2
        pltpu.sync_copy(dest_hbm.at[pl.ds(tbase, CH2)], d0)
        pltpu.sync_copy(dest_hbm.at[pl.ds(N + tbase, CH2)], d1)
        cp1 = pltpu.async_copy(y_hbm.at[d0], r0, sem1)
        cp2 = pltpu.async_copy(y_hbm.at[d1], r1, sem2)
        cp1.wait()
        cp2.wait()

        def cbody(t, _):
            for kk in range(DIM // L):
                sl = pl.ds(kk * L, L)
                ob[t, sl] = r0[t, sl] + r1[t, sl]
            return 0

        lax.fori_loop(0, CH2, cbody, 0)
        pltpu.sync_copy(ob, out_hbm.at[pl.ds(tbase, CH2)])


def kernel(x, Wr, W1, W2):
    Bb, Tt, D = x.shape
    x_flat = x.reshape(N, D)
    a1, a2, w1v, w2v, c1, c2 = _router(x_flat, Wr)
    eflat = jnp.concatenate([a1, a2])
    wq = jnp.broadcast_to(jnp.concatenate([w1v, w2v])[:, None], (NK, 128))
    cnt = jnp.concatenate([c1.reshape(NW // 2, L), c2.reshape(NW // 2, L)])
    dest, xp, sw, beo, beu = _dispatch_fn()(eflat, wq, cnt, x_flat)
    y = _ffn(beo, beu, xp, sw,
             W1.astype(jnp.bfloat16), W2.astype(jnp.bfloat16))
    out = _combine_fn()(y, dest)
    return out.reshape(Bb, Tt, D)


# speedup vs baseline: 1.5089x; 1.0296x over previous
import functools

import jax
import jax.numpy as jnp
from jax import lax
from jax.experimental import pallas as pl
from jax.experimental.pallas import tpu as pltpu
from jax.experimental.pallas import tpu_sc as plsc

DIM = 1024
FF = 4096
E = 8
K = 2
N = 4096
NK = N * K
NC, NS, L = 2, 16, 16
NW = NC * NS
S = NK // NW
TPT = N // NW
BLK = 512
BSH = 9
P = NK + E * BLK
RB = P // BLK
FFC = 1024
NFF = FF // FFC
RN = 1024
CH = 32
CH2 = 16

@functools.cache
def _mesh():
    return plsc.VectorSubcoreMesh(
        core_axis_name="c", subcore_axis_name="s",
        num_cores=NC, num_subcores=NS)


def _gather16(src, idx):
    return lax.gather(
        src,
        idx[:, None],
        lax.GatherDimensionNumbers(
            offset_dims=(), collapsed_slice_dims=(0,), start_index_map=(0,)),
        (1,),
        mode=lax.GatherScatterMode.PROMISE_IN_BOUNDS,
    )


def _router_body(x_ref, wr_ref, a1_ref, a2_ref, w1_ref, w2_ref, c1_ref, c2_ref):
    xb = x_ref[...]
    logits = jax.lax.dot_general(
        xb, wr_ref[...], (((1,), (1,)), ((), ())),
        preferred_element_type=jnp.float32)
    ids = lax.broadcasted_iota(jnp.int32, (RN, E), 1)
    m1 = jnp.max(logits, axis=1, keepdims=True)
    a1 = jnp.min(jnp.where(logits == m1, ids, E), axis=1)
    neg = jnp.finfo(jnp.float32).min
    l2 = jnp.where(ids == a1[:, None], neg, logits)
    m2 = jnp.max(l2, axis=1, keepdims=True)
    a2 = jnp.min(jnp.where(l2 == m2, ids, E), axis=1)
    g = 1.0 / (1.0 + jnp.exp(m2[:, 0] - m1[:, 0]))
    a1_ref[...] = a1
    a2_ref[...] = a2
    w1_ref[...] = g
    w2_ref[...] = 1.0 - g
    ids16 = lax.broadcasted_iota(jnp.int32, (RN, L), 1)
    grp = (lax.broadcasted_iota(jnp.int32, (RN // S, RN), 1) // S ==
           lax.broadcasted_iota(jnp.int32, (RN // S, RN), 0)).astype(jnp.float32)
    oh1 = (ids16 == a1[:, None]).astype(jnp.float32)
    oh2 = (ids16 == a2[:, None]).astype(jnp.float32)
    c1_ref[...] = jnp.dot(grp, oh1, preferred_element_type=jnp.float32
                          ).astype(jnp.int32).reshape(1, RN // S, L)
    c2_ref[...] = jnp.dot(grp, oh2, preferred_element_type=jnp.float32
                          ).astype(jnp.int32).reshape(1, RN // S, L)


def _router(x_flat, Wr):
    nblk = N // RN
    return pl.pallas_call(
        _router_body,
        grid=(nblk,),
        in_specs=[
            pl.BlockSpec((RN, DIM), lambda b: (b, 0)),
            pl.BlockSpec((E, DIM), lambda b: (0, 0)),
        ],
        out_specs=[
            pl.BlockSpec((RN,), lambda b: (b,)),
            pl.BlockSpec((RN,), lambda b: (b,)),
            pl.BlockSpec((RN,), lambda b: (b,)),
            pl.BlockSpec((RN,), lambda b: (b,)),
            pl.BlockSpec((1, RN // S, L), lambda b: (b, 0, 0)),
            pl.BlockSpec((1, RN // S, L), lambda b: (b, 0, 0)),
        ],
        out_shape=[
            jax.ShapeDtypeStruct((N,), jnp.int32),
            jax.ShapeDtypeStruct((N,), jnp.int32),
            jax.ShapeDtypeStruct((N,), jnp.float32),
            jax.ShapeDtypeStruct((N,), jnp.float32),
            jax.ShapeDtypeStruct((nblk, RN // S, L), jnp.int32),
            jax.ShapeDtypeStruct((nblk, RN // S, L), jnp.int32),
        ],
    )(x_flat, Wr)


@functools.cache
def _dispatch_fn():
    return functools.partial(
        pl.kernel,
        out_type=[
            jax.ShapeDtypeStruct((NK,), jnp.int32),
            jax.ShapeDtypeStruct((P, DIM), jnp.float32),
            jax.ShapeDtypeStruct((RB,), jnp.int32),
            jax.ShapeDtypeStruct((RB,), jnp.int32),
        ],
        mesh=_mesh(),
        scratch_types=[
            pltpu.VMEM((NW, L), jnp.int32),
            pltpu.VMEM((S,), jnp.int32),
            pltpu.VMEM((S // CH, CH), jnp.int32),
            pltpu.VMEM((S,), jnp.int32),
            pltpu.VMEM((2, CH, DIM), jnp.float32),
            pltpu.VMEM((-(-RB // L) * L,), jnp.int32),
            pltpu.VMEM((-(-RB // L) * L,), jnp.int32),
            pltpu.SemaphoreType.DMA,
            pltpu.SemaphoreType.DMA,
            pltpu.SemaphoreType.DMA,
            pltpu.SemaphoreType.DMA,
        ],
        compiler_params=pltpu.CompilerParams(needs_layout_passes=False),
    )(_dispatch_body)


def _dispatch_body(e_hbm, cnt_hbm, x_hbm,
                   dest_hbm, xp_hbm, beo_hbm, beu_hbm,
                   cnt_v, ev, destv, destf, xr2, beov, beuv,
                   sl0, sl1, ss0, ss1):
    wid = lax.axis_index("s") * NC + lax.axis_index("c")
    pltpu.sync_copy(cnt_hbm, cnt_v)
    pltpu.sync_copy(e_hbm.at[pl.ds(wid * S, S)], ev)

    lane = lax.broadcasted_iota(jnp.int32, (L,), 0)
    zero = jnp.zeros((L,), jnp.int32)
    tot = zero
    pre = zero
    for j in range(NW):
        row = cnt_v[j]
        tot = tot + row
        pre = pre + jnp.where(jnp.full((L,), j, jnp.int32) < wid, row, zero)
    padded = ((tot + (BLK - 1)) >> BSH) << BSH
    incl = plsc.cumsum(padded)
    po = incl - padded
    base = po + pre

    nv_per_row = CH // L

    def dbody(j, run):
        v = ev[pl.ds(j * L, L)]
        rank = zero
        hist = zero
        for e in range(E):
            m = v == e
            mi = m.astype(jnp.int32)
            inc = plsc.cumsum(mi)
            rank = jnp.where(m, inc - 1, rank)
            hist = jnp.where(lane == e, jnp.sum(mi), hist)
        dvec = _gather16(base + run, v) + rank
        destv[j // nv_per_row, pl.ds((j % nv_per_row) * L, L)] = dvec
        destf[pl.ds(j * L, L)] = dvec
        return run + hist

    lax.fori_loop(0, S // L, dbody, zero)

    pltpu.sync_copy(destf, dest_hbm.at[pl.ds(wid * S, S)])

    tb = jnp.where(wid < NW // 2, wid * S, wid * S - N)
    nch = S // CH
    sl = (sl0, sl1)
    ss = (ss0, ss1)

    def _load(c, b):
        return pltpu.async_copy(
            x_hbm.at[pl.ds(tb + c * CH, CH)], xr2.at[b], sl[b])

    def _scat(c, b):
        return pltpu.async_copy(xr2.at[b], xp_hbm.at[destv.at[c]], ss[b])

    loads = {0: _load(0, 0)}
    scats = {}
    for c in range(nch):
        b = c & 1
        loads[c].wait()
        if c + 1 < nch:
            if c - 1 >= 0:
                scats[c - 1].wait()
            loads[c + 1] = _load(c + 1, 1 - b)
        scats[c] = _scat(c, b)
    scats[nch - 2].wait()
    scats[nch - 1].wait()

    @pl.when(wid == 0)
    def _():
        for jb in range(-(-RB // L)):
            bstart = (lane + jb * L) * BLK
            acc = zero
            for e in range(E):
                th = _gather16(po, jnp.full((L,), e, jnp.int32))
                acc = acc + jnp.where(th <= bstart, 1, 0)
            bev = acc - 1
            beov[pl.ds(jb * L, L)] = bev
            rend = _gather16(po + tot, bev)
            beuv[pl.ds(jb * L, L)] = jnp.where(bstart < rend, 1, 0)
        pltpu.sync_copy(beov.at[pl.ds(0, RB)], beo_hbm)
        pltpu.sync_copy(beuv.at[pl.ds(0, RB)], beu_hbm)


def _ffn_body(be_ref, bu_ref, xp_ref, w1_ref, w2_ref, y_ref, acc):
    rb = pl.program_id(0)
    fc = pl.program_id(1)

    @pl.when(bu_ref[rb] == 1)
    def _():
        h = jnp.dot(xp_ref[...].astype(jnp.bfloat16), w1_ref[0],
                    preferred_element_type=jnp.float32)
        h = 0.5 * h * (1.0 + lax.erf(h * (2.0 ** -0.5)))
        p = jnp.dot(h.astype(jnp.bfloat16), w2_ref[0],
                    preferred_element_type=jnp.float32)

        @pl.when(fc == 0)
        def _():
            acc[...] = p

        @pl.when(fc > 0)
        def _():
            acc[...] += p

        @pl.when(fc == NFF - 1)
        def _():
            y_ref[...] = acc[...]


def _ffn(beo, beu, xp, W1, W2):
    grid_spec = pltpu.PrefetchScalarGridSpec(
        num_scalar_prefetch=2,
        grid=(RB, NFF),
        in_specs=[
            pl.BlockSpec((BLK, DIM), lambda rb, fc, be, bu: (rb * bu[rb], 0)),
            pl.BlockSpec((1, DIM, FFC),
                         lambda rb, fc, be, bu: (be[rb], 0, fc * bu[rb])),
            pl.BlockSpec((1, FFC, DIM),
                         lambda rb, fc, be, bu: (be[rb], fc * bu[rb], 0)),
        ],
        out_specs=pl.BlockSpec((BLK, DIM), lambda rb, fc, be, bu: (rb, 0)),
        scratch_shapes=[pltpu.VMEM((BLK, DIM), jnp.float32)],
    )
    return pl.pallas_call(
        _ffn_body,
        grid_spec=grid_spec,
        out_shape=jax.ShapeDtypeStruct((P, DIM), jnp.float32),
        compiler_params=pltpu.CompilerParams(
            dimension_semantics=("arbitrary", "arbitrary")),
    )(beo, beu, xp, W1, W2)


@functools.cache
def _combine_fn():
    return functools.partial(
        pl.kernel,
        out_type=jax.ShapeDtypeStruct((N, DIM), jnp.float32),
        mesh=_mesh(),
        scratch_types=[
            pltpu.VMEM((TPT,), jnp.int32),
            pltpu.VMEM((TPT,), jnp.int32),
            pltpu.VMEM((TPT,), jnp.float32),
            pltpu.VMEM((TPT,), jnp.float32),
            pltpu.VMEM((2, CH2, DIM), jnp.float32),
            pltpu.VMEM((2, CH2, DIM), jnp.float32),
            pltpu.VMEM((2, CH2, DIM), jnp.float32),
            pltpu.SemaphoreType.DMA,
            pltpu.SemaphoreType.DMA,
            pltpu.SemaphoreType.DMA,
            pltpu.SemaphoreType.DMA,
            pltpu.SemaphoreType.DMA,
            pltpu.SemaphoreType.DMA,
        ],
        compiler_params=pltpu.CompilerParams(needs_layout_passes=False),
    )(_combine_body)


def _combine_body(y_hbm, dest_hbm, w_hbm, out_hbm,
                  d0v, d1v, w0v, w1v, r0, r1, ob,
                  g00, g01, g10, g11, so0, so1):
    wid = lax.axis_index("s") * NC + lax.axis_index("c")
    t0 = wid * TPT
    pltpu.sync_copy(dest_hbm.at[pl.ds(t0, TPT)], d0v)
    pltpu.sync_copy(dest_hbm.at[pl.ds(N + t0, TPT)], d1v)
    pltpu.sync_copy(w_hbm.at[pl.ds(t0, TPT)], w0v)
    pltpu.sync_copy(w_hbm.at[pl.ds(N + t0, TPT)], w1v)

    nch = TPT // CH2
    g0s = (g00, g01)
    g1s = (g10, g11)
    sos = (so0, so1)

    def _gath(c, b):
        return (pltpu.async_copy(
                    y_hbm.at[d0v.at[pl.ds(c * CH2, CH2)]], r0.at[b], g0s[b]),
                pltpu.async_copy(
                    y_hbm.at[d1v.at[pl.ds(c * CH2, CH2)]], r1.at[b], g1s[b]))

    def _store(c, b):
        return pltpu.async_copy(
            ob.at[b], out_hbm.at[pl.ds(t0 + c * CH2, CH2)], sos[b])

    gats = {0: _gath(0, 0)}
    stores = {}
    for c in range(nch):
        b = c & 1
        ga, gb = gats[c]
        ga.wait()
        gb.wait()
        if c + 1 < nch:
            gats[c + 1] = _gath(c + 1, 1 - b)
        if c - 2 >= 0:
            stores[c - 2].wait()

        wv0 = w0v[pl.ds(c * CH2, CH2)]
        wv1 = w1v[pl.ds(c * CH2, CH2)]

        def cbody(t, _):
            tt = jnp.full((L,), t, jnp.int32)
            w0s = _gather16(wv0, tt)
            w1s = _gather16(wv1, tt)
            for kk in range(DIM // L):
                sl = pl.ds(kk * L, L)
                ob[b, t, sl] = w0s * r0[b, t, sl] + w1s * r1[b, t, sl]
            return 0

        lax.fori_loop(0, CH2, cbody, 0)
        stores[c] = _store(c, b)
    stores[nch - 2].wait()
    stores[nch - 1].wait()


def kernel(x, Wr, W1, W2):
    Bb, Tt, D = x.shape
    x_flat = x.reshape(N, D)
    a1, a2, w1v, w2v, c1, c2 = _router(x_flat, Wr)
    eflat = jnp.concatenate([a1, a2])
    wflat = jnp.concatenate([w1v, w2v])
    cnt = jnp.concatenate([c1.reshape(NW // 2, L), c2.reshape(NW // 2, L)])
    dest, xp, beo, beu = _dispatch_fn()(eflat, cnt, x_flat)
    y = _ffn(beo, beu, xp,
             W1.astype(jnp.bfloat16), W2.astype(jnp.bfloat16))
    out = _combine_fn()(y, dest, wflat)
    return out.reshape(Bb, Tt, D)


# speedup vs baseline: 1.5120x; 1.0020x over previous
import functools

import jax
import jax.numpy as jnp
from jax import lax
from jax.experimental import pallas as pl
from jax.experimental.pallas import tpu as pltpu
from jax.experimental.pallas import tpu_sc as plsc

DIM = 1024
FF = 4096
E = 8
K = 2
N = 4096
NK = N * K
NC, NS, L = 2, 16, 16
NW = NC * NS
S = NK // NW
TPT = N // NW
BLK = 512
BSH = 9
P = NK + E * BLK
RB = P // BLK
FFC = 1024
NFF = FF // FFC
RN = 1024
CH = 32
CH2 = 16

@functools.cache
def _mesh():
    return plsc.VectorSubcoreMesh(
        core_axis_name="c", subcore_axis_name="s",
        num_cores=NC, num_subcores=NS)


def _gather16(src, idx):
    return lax.gather(
        src,
        idx[:, None],
        lax.GatherDimensionNumbers(
            offset_dims=(), collapsed_slice_dims=(0,), start_index_map=(0,)),
        (1,),
        mode=lax.GatherScatterMode.PROMISE_IN_BOUNDS,
    )


def _router_body(x_ref, wr_ref, e2_ref, w2_ref, cc_ref):
    xb = x_ref[...]
    logits = jax.lax.dot_general(
        xb, wr_ref[...], (((1,), (1,)), ((), ())),
        preferred_element_type=jnp.float32)
    ids = lax.broadcasted_iota(jnp.int32, (RN, E), 1)
    m1 = jnp.max(logits, axis=1, keepdims=True)
    a1 = jnp.min(jnp.where(logits == m1, ids, E), axis=1)
    neg = jnp.finfo(jnp.float32).min
    l2 = jnp.where(ids == a1[:, None], neg, logits)
    m2 = jnp.max(l2, axis=1, keepdims=True)
    a2 = jnp.min(jnp.where(l2 == m2, ids, E), axis=1)
    g = 1.0 / (1.0 + jnp.exp(m2[:, 0] - m1[:, 0]))
    e2_ref[0] = a1
    e2_ref[1] = a2
    w2_ref[0] = g
    w2_ref[1] = 1.0 - g
    ids16 = lax.broadcasted_iota(jnp.int32, (RN, L), 1)
    grp = (lax.broadcasted_iota(jnp.int32, (RN // S, RN), 1) // S ==
           lax.broadcasted_iota(jnp.int32, (RN // S, RN), 0)).astype(jnp.float32)
    oh1 = (ids16 == a1[:, None]).astype(jnp.float32)
    oh2 = (ids16 == a2[:, None]).astype(jnp.float32)
    cc_ref[0] = jnp.dot(grp, oh1, preferred_element_type=jnp.float32
                        ).astype(jnp.int32).reshape(1, RN // S, L)
    cc_ref[1] = jnp.dot(grp, oh2, preferred_element_type=jnp.float32
                        ).astype(jnp.int32).reshape(1, RN // S, L)


def _router(x_flat, Wr):
    nblk = N // RN
    return pl.pallas_call(
        _router_body,
        grid=(nblk,),
        in_specs=[
            pl.BlockSpec((RN, DIM), lambda b: (b, 0)),
            pl.BlockSpec((E, DIM), lambda b: (0, 0)),
        ],
        out_specs=[
            pl.BlockSpec((2, RN), lambda b: (0, b)),
            pl.BlockSpec((2, RN), lambda b: (0, b)),
            pl.BlockSpec((2, 1, RN // S, L), lambda b: (0, b, 0, 0)),
        ],
        out_shape=[
            jax.ShapeDtypeStruct((2, N), jnp.int32),
            jax.ShapeDtypeStruct((2, N), jnp.float32),
            jax.ShapeDtypeStruct((2, nblk, RN // S, L), jnp.int32),
        ],
    )(x_flat, Wr)


@functools.cache
def _dispatch_fn():
    return functools.partial(
        pl.kernel,
        out_type=[
            jax.ShapeDtypeStruct((NK,), jnp.int32),
            jax.ShapeDtypeStruct((P, DIM), jnp.float32),
            jax.ShapeDtypeStruct((RB,), jnp.int32),
            jax.ShapeDtypeStruct((RB,), jnp.int32),
        ],
        mesh=_mesh(),
        scratch_types=[
            pltpu.VMEM((NW, L), jnp.int32),
            pltpu.VMEM((S,), jnp.int32),
            pltpu.VMEM((S // CH, CH), jnp.int32),
            pltpu.VMEM((S,), jnp.int32),
            pltpu.VMEM((2, CH, DIM), jnp.float32),
            pltpu.VMEM((-(-RB // L) * L,), jnp.int32),
            pltpu.VMEM((-(-RB // L) * L,), jnp.int32),
            pltpu.SemaphoreType.DMA,
            pltpu.SemaphoreType.DMA,
            pltpu.SemaphoreType.DMA,
            pltpu.SemaphoreType.DMA,
        ],
        compiler_params=pltpu.CompilerParams(needs_layout_passes=False),
    )(_dispatch_body)


def _dispatch_body(e_hbm, cnt_hbm, x_hbm,
                   dest_hbm, xp_hbm, beo_hbm, beu_hbm,
                   cnt_v, ev, destv, destf, xr2, beov, beuv,
                   sl0, sl1, ss0, ss1):
    wid = lax.axis_index("s") * NC + lax.axis_index("c")
    pltpu.sync_copy(cnt_hbm, cnt_v)
    pltpu.sync_copy(e_hbm.at[pl.ds(wid * S, S)], ev)

    lane = lax.broadcasted_iota(jnp.int32, (L,), 0)
    zero = jnp.zeros((L,), jnp.int32)
    tot = zero
    pre = zero
    for j in range(NW):
        row = cnt_v[j]
        tot = tot + row
        pre = pre + jnp.where(jnp.full((L,), j, jnp.int32) < wid, row, zero)
    padded = ((tot + (BLK - 1)) >> BSH) << BSH
    incl = plsc.cumsum(padded)
    po = incl - padded
    base = po + pre

    nv_per_row = CH // L

    def dbody(j, run):
        v = ev[pl.ds(j * L, L)]
        rank = zero
        hist = zero
        for e in range(E):
            m = v == e
            mi = m.astype(jnp.int32)
            inc = plsc.cumsum(mi)
            rank = jnp.where(m, inc - 1, rank)
            hist = jnp.where(lane == e, jnp.sum(mi), hist)
        dvec = _gather16(base + run, v) + rank
        destv[j // nv_per_row, pl.ds((j % nv_per_row) * L, L)] = dvec
        destf[pl.ds(j * L, L)] = dvec
        return run + hist

    lax.fori_loop(0, S // L, dbody, zero)

    pltpu.sync_copy(destf, dest_hbm.at[pl.ds(wid * S, S)])

    tb = jnp.where(wid < NW // 2, wid * S, wid * S - N)
    nch = S // CH
    sl = (sl0, sl1)
    ss = (ss0, ss1)

    def _load(c, b):
        return pltpu.async_copy(
            x_hbm.at[pl.ds(tb + c * CH, CH)], xr2.at[b], sl[b])

    def _scat(c, b):
        return pltpu.async_copy(xr2.at[b], xp_hbm.at[destv.at[c]], ss[b])

    loads = {0: _load(0, 0)}
    scats = {}
    for c in range(nch):
        b = c & 1
        loads[c].wait()
        if c + 1 < nch:
            if c - 1 >= 0:
                scats[c - 1].wait()
            loads[c + 1] = _load(c + 1, 1 - b)
        scats[c] = _scat(c, b)
    scats[nch - 2].wait()
    scats[nch - 1].wait()

    @pl.when(wid == 0)
    def _():
        for jb in range(-(-RB // L)):
            bstart = (lane + jb * L) * BLK
            acc = zero
            for e in range(E):
                th = _gather16(po, jnp.full((L,), e, jnp.int32))
                acc = acc + jnp.where(th <= bstart, 1, 0)
            bev = acc - 1
            beov[pl.ds(jb * L, L)] = bev
            rend = _gather16(po + tot, bev)
            beuv[pl.ds(jb * L, L)] = jnp.where(bstart < rend, 1, 0)
        pltpu.sync_copy(beov.at[pl.ds(0, RB)], beo_hbm)
        pltpu.sync_copy(beuv.at[pl.ds(0, RB)], beu_hbm)


def _ffn_body(be_ref, bu_ref, xp_ref, w1_ref, w2_ref, y_ref, acc):
    rb = pl.program_id(0)
    fc = pl.program_id(1)

    @pl.when(bu_ref[rb] == 1)
    def _():
        h = jnp.dot(xp_ref[...].astype(jnp.bfloat16), w1_ref[0],
                    preferred_element_type=jnp.float32)
        h = 0.5 * h * (1.0 + lax.erf(h * (2.0 ** -0.5)))
        p = jnp.dot(h.astype(jnp.bfloat16), w2_ref[0],
                    preferred_element_type=jnp.float32)

        @pl.when(fc == 0)
        def _():
            acc[...] = p

        @pl.when(fc > 0)
        def _():
            acc[...] += p

        @pl.when(fc == NFF - 1)
        def _():
            y_ref[...] = acc[...]


def _ffn(beo, beu, xp, W1, W2):
    grid_spec = pltpu.PrefetchScalarGridSpec(
        num_scalar_prefetch=2,
        grid=(RB, NFF),
        in_specs=[
            pl.BlockSpec((BLK, DIM), lambda rb, fc, be, bu: (rb * bu[rb], 0)),
            pl.BlockSpec((1, DIM, FFC),
                         lambda rb, fc, be, bu: (be[rb], 0, fc * bu[rb])),
            pl.BlockSpec((1, FFC, DIM),
                         lambda rb, fc, be, bu: (be[rb], fc * bu[rb], 0)),
        ],
        out_specs=pl.BlockSpec((BLK, DIM), lambda rb, fc, be, bu: (rb, 0)),
        scratch_shapes=[pltpu.VMEM((BLK, DIM), jnp.float32)],
    )
    return pl.pallas_call(
        _ffn_body,
        grid_spec=grid_spec,
        out_shape=jax.ShapeDtypeStruct((P, DIM), jnp.float32),
        compiler_params=pltpu.CompilerParams(
            dimension_semantics=("arbitrary", "arbitrary")),
    )(beo, beu, xp, W1, W2)


@functools.cache
def _combine_fn():
    return functools.partial(
        pl.kernel,
        out_type=jax.ShapeDtypeStruct((N, DIM), jnp.float32),
        mesh=_mesh(),
        scratch_types=[
            pltpu.VMEM((TPT,), jnp.int32),
            pltpu.VMEM((TPT,), jnp.int32),
            pltpu.VMEM((TPT,), jnp.float32),
            pltpu.VMEM((TPT,), jnp.float32),
            pltpu.VMEM((2, CH2, DIM), jnp.float32),
            pltpu.VMEM((2, CH2, DIM), jnp.float32),
            pltpu.VMEM((2, CH2, DIM), jnp.float32),
            pltpu.SemaphoreType.DMA,
            pltpu.SemaphoreType.DMA,
            pltpu.SemaphoreType.DMA,
            pltpu.SemaphoreType.DMA,
            pltpu.SemaphoreType.DMA,
            pltpu.SemaphoreType.DMA,
        ],
        compiler_params=pltpu.CompilerParams(needs_layout_passes=False),
    )(_combine_body)


def _combine_body(y_hbm, dest_hbm, w_hbm, out_hbm,
                  d0v, d1v, w0v, w1v, r0, r1, ob,
                  g00, g01, g10, g11, so0, so1):
    wid = lax.axis_index("s") * NC + lax.axis_index("c")
    t0 = wid * TPT
    pltpu.sync_copy(dest_hbm.at[pl.ds(t0, TPT)], d0v)
    pltpu.sync_copy(dest_hbm.at[pl.ds(N + t0, TPT)], d1v)
    pltpu.sync_copy(w_hbm.at[pl.ds(t0, TPT)], w0v)
    pltpu.sync_copy(w_hbm.at[pl.ds(N + t0, TPT)], w1v)

    nch = TPT // CH2
    g0s = (g00, g01)
    g1s = (g10, g11)
    sos = (so0, so1)

    def _gath(c, b):
        return (pltpu.async_copy(
                    y_hbm.at[d0v.at[pl.ds(c * CH2, CH2)]], r0.at[b], g0s[b]),
                pltpu.async_copy(
                    y_hbm.at[d1v.at[pl.ds(c * CH2, CH2)]], r1.at[b], g1s[b]))

    def _store(c, b):
        return pltpu.async_copy(
            ob.at[b], out_hbm.at[pl.ds(t0 + c * CH2, CH2)], sos[b])

    gats = {0: _gath(0, 0)}
    stores = {}
    for c in range(nch):
        b = c & 1
        ga, gb = gats[c]
        ga.wait()
        gb.wait()
        if c + 1 < nch:
            gats[c + 1] = _gath(c + 1, 1 - b)
        if c - 2 >= 0:
            stores[c - 2].wait()

        wv0 = w0v[pl.ds(c * CH2, CH2)]
        wv1 = w1v[pl.ds(c * CH2, CH2)]

        def cbody(t, _):
            tt = jnp.full((L,), t, jnp.int32)
            w0s = _gather16(wv0, tt)
            w1s = _gather16(wv1, tt)
            for kk in range(DIM // L):
                sl = pl.ds(kk * L, L)
                ob[b, t, sl] = w0s * r0[b, t, sl] + w1s * r1[b, t, sl]
            return 0

        lax.fori_loop(0, CH2, cbody, 0)
        stores[c] = _store(c, b)
    stores[nch - 2].wait()
    stores[nch - 1].wait()


def kernel(x, Wr, W1, W2):
    Bb, Tt, D = x.shape
    x_flat = x.reshape(N, D)
    e2, w2, cc = _router(x_flat, Wr)
    eflat = e2.reshape(NK)
    wflat = w2.reshape(NK)
    cnt = cc.reshape(NW, L)
    dest, xp, beo, beu = _dispatch_fn()(eflat, cnt, x_flat)
    y = _ffn(beo, beu, xp,
             W1.astype(jnp.bfloat16), W2.astype(jnp.bfloat16))
    out = _combine_fn()(y, dest, wflat)
    return out.reshape(Bb, Tt, D)


# speedup vs baseline: 1.6029x; 1.0601x over previous
import functools

import jax
import jax.numpy as jnp
from jax import lax
from jax.experimental import pallas as pl
from jax.experimental.pallas import tpu as pltpu
from jax.experimental.pallas import tpu_sc as plsc

DIM = 1024
FF = 4096
E = 8
K = 2
N = 4096
NK = N * K
NC, NS, L = 2, 16, 16
NW = NC * NS
S = NK // NW
TPT = N // NW
BLK = 512
BSH = 9
P = NK + E * BLK
RB = P // BLK
FFC = 2048
NFF = FF // FFC
RN = 1024
CH = 32
CH2 = 16

@functools.cache
def _mesh():
    return plsc.VectorSubcoreMesh(
        core_axis_name="c", subcore_axis_name="s",
        num_cores=NC, num_subcores=NS)


def _gather16(src, idx):
    return lax.gather(
        src,
        idx[:, None],
        lax.GatherDimensionNumbers(
            offset_dims=(), collapsed_slice_dims=(0,), start_index_map=(0,)),
        (1,),
        mode=lax.GatherScatterMode.PROMISE_IN_BOUNDS,
    )


def _router_body(x_ref, wr_ref, e2_ref, w2_ref, cc_ref):
    xb = x_ref[...]
    logits = jax.lax.dot_general(
        xb, wr_ref[...], (((1,), (1,)), ((), ())),
        preferred_element_type=jnp.float32)
    ids = lax.broadcasted_iota(jnp.int32, (RN, E), 1)
    m1 = jnp.max(logits, axis=1, keepdims=True)
    a1 = jnp.min(jnp.where(logits == m1, ids, E), axis=1)
    neg = jnp.finfo(jnp.float32).min
    l2 = jnp.where(ids == a1[:, None], neg, logits)
    m2 = jnp.max(l2, axis=1, keepdims=True)
    a2 = jnp.min(jnp.where(l2 == m2, ids, E), axis=1)
    g = 1.0 / (1.0 + jnp.exp(m2[:, 0] - m1[:, 0]))
    e2_ref[0] = a1
    e2_ref[1] = a2
    w2_ref[0] = g
    w2_ref[1] = 1.0 - g
    ids16 = lax.broadcasted_iota(jnp.int32, (RN, L), 1)
    grp = (lax.broadcasted_iota(jnp.int32, (RN // S, RN), 1) // S ==
           lax.broadcasted_iota(jnp.int32, (RN // S, RN), 0)).astype(jnp.float32)
    oh1 = (ids16 == a1[:, None]).astype(jnp.float32)
    oh2 = (ids16 == a2[:, None]).astype(jnp.float32)
    cc_ref[0] = jnp.dot(grp, oh1, preferred_element_type=jnp.float32
                        ).astype(jnp.int32).reshape(1, RN // S, L)
    cc_ref[1] = jnp.dot(grp, oh2, preferred_element_type=jnp.float32
                        ).astype(jnp.int32).reshape(1, RN // S, L)


def _router(x_flat, Wr):
    nblk = N // RN
    return pl.pallas_call(
        _router_body,
        grid=(nblk,),
        in_specs=[
            pl.BlockSpec((RN, DIM), lambda b: (b, 0)),
            pl.BlockSpec((E, DIM), lambda b: (0, 0)),
        ],
        out_specs=[
            pl.BlockSpec((2, RN), lambda b: (0, b)),
            pl.BlockSpec((2, RN), lambda b: (0, b)),
            pl.BlockSpec((2, 1, RN // S, L), lambda b: (0, b, 0, 0)),
        ],
        out_shape=[
            jax.ShapeDtypeStruct((2, N), jnp.int32),
            jax.ShapeDtypeStruct((2, N), jnp.float32),
            jax.ShapeDtypeStruct((2, nblk, RN // S, L), jnp.int32),
        ],
    )(x_flat, Wr)


@functools.cache
def _dispatch_fn():
    return functools.partial(
        pl.kernel,
        out_type=[
            jax.ShapeDtypeStruct((NK,), jnp.int32),
            jax.ShapeDtypeStruct((P, DIM), jnp.float32),
            jax.ShapeDtypeStruct((RB,), jnp.int32),
            jax.ShapeDtypeStruct((RB,), jnp.int32),
        ],
        mesh=_mesh(),
        scratch_types=[
            pltpu.VMEM((NW, L), jnp.int32),
            pltpu.VMEM((S,), jnp.int32),
            pltpu.VMEM((S // CH, CH), jnp.int32),
            pltpu.VMEM((S,), jnp.int32),
            pltpu.VMEM((2, CH, DIM), jnp.float32),
            pltpu.VMEM((-(-RB // L) * L,), jnp.int32),
            pltpu.VMEM((-(-RB // L) * L,), jnp.int32),
            pltpu.SemaphoreType.DMA,
            pltpu.SemaphoreType.DMA,
            pltpu.SemaphoreType.DMA,
            pltpu.SemaphoreType.DMA,
        ],
        compiler_params=pltpu.CompilerParams(needs_layout_passes=False),
    )(_dispatch_body)


def _dispatch_body(e_hbm, cnt_hbm, x_hbm,
                   dest_hbm, xp_hbm, beo_hbm, beu_hbm,
                   cnt_v, ev, destv, destf, xr2, beov, beuv,
                   sl0, sl1, ss0, ss1):
    wid = lax.axis_index("s") * NC + lax.axis_index("c")
    pltpu.sync_copy(cnt_hbm, cnt_v)
    pltpu.sync_copy(e_hbm.at[pl.ds(wid * S, S)], ev)

    lane = lax.broadcasted_iota(jnp.int32, (L,), 0)
    zero = jnp.zeros((L,), jnp.int32)
    tot = zero
    pre = zero
    for j in range(NW):
        row = cnt_v[j]
        tot = tot + row
        pre = pre + jnp.where(jnp.full((L,), j, jnp.int32) < wid, row, zero)
    padded = ((tot + (BLK - 1)) >> BSH) << BSH
    incl = plsc.cumsum(padded)
    po = incl - padded
    base = po + pre

    nv_per_row = CH // L

    def dbody(j, run):
        v = ev[pl.ds(j * L, L)]
        rank = zero
        hist = zero
        for e in range(E):
            m = v == e
            mi = m.astype(jnp.int32)
            inc = plsc.cumsum(mi)
            rank = jnp.where(m, inc - 1, rank)
            hist = jnp.where(lane == e, jnp.sum(mi), hist)
        dvec = _gather16(base + run, v) + rank
        destv[j // nv_per_row, pl.ds((j % nv_per_row) * L, L)] = dvec
        destf[pl.ds(j * L, L)] = dvec
        return run + hist

    lax.fori_loop(0, S // L, dbody, zero)

    pltpu.sync_copy(destf, dest_hbm.at[pl.ds(wid * S, S)])

    tb = jnp.where(wid < NW // 2, wid * S, wid * S - N)
    nch = S // CH
    sl = (sl0, sl1)
    ss = (ss0, ss1)

    def _load(c, b):
        return pltpu.async_copy(
            x_hbm.at[pl.ds(tb + c * CH, CH)], xr2.at[b], sl[b])

    def _scat(c, b):
        return pltpu.async_copy(xr2.at[b], xp_hbm.at[destv.at[c]], ss[b])

    loads = {0: _load(0, 0)}
    scats = {}
    for c in range(nch):
        b = c & 1
        loads[c].wait()
        if c + 1 < nch:
            if c - 1 >= 0:
                scats[c - 1].wait()
            loads[c + 1] = _load(c + 1, 1 - b)
        scats[c] = _scat(c, b)
    scats[nch - 2].wait()
    scats[nch - 1].wait()

    @pl.when(wid == 0)
    def _():
        for jb in range(-(-RB // L)):
            bstart = (lane + jb * L) * BLK
            acc = zero
            for e in range(E):
                th = _gather16(po, jnp.full((L,), e, jnp.int32))
                acc = acc + jnp.where(th <= bstart, 1, 0)
            bev = acc - 1
            beov[pl.ds(jb * L, L)] = bev
            rend = _gather16(po + tot, bev)
            beuv[pl.ds(jb * L, L)] = jnp.where(bstart < rend, 1, 0)
        pltpu.sync_copy(beov.at[pl.ds(0, RB)], beo_hbm)
        pltpu.sync_copy(beuv.at[pl.ds(0, RB)], beu_hbm)


def _ffn_body(be_ref, bu_ref, xp_ref, w1_ref, w2_ref, y_ref, acc):
    rb = pl.program_id(0)
    fc = pl.program_id(1)

    @pl.when(bu_ref[rb] == 1)
    def _():
        h = jnp.dot(xp_ref[...].astype(jnp.bfloat16), w1_ref[0],
                    preferred_element_type=jnp.float32)
        h = 0.5 * h * (1.0 + lax.erf(h * (2.0 ** -0.5)))
        p = jnp.dot(h.astype(jnp.bfloat16), w2_ref[0],
                    preferred_element_type=jnp.float32)

        @pl.when(fc == 0)
        def _():
            acc[...] = p

        @pl.when(fc > 0)
        def _():
            acc[...] += p

        @pl.when(fc == NFF - 1)
        def _():
            y_ref[...] = acc[...]


def _ffn(beo, beu, xp, W1, W2):
    grid_spec = pltpu.PrefetchScalarGridSpec(
        num_scalar_prefetch=2,
        grid=(RB, NFF),
        in_specs=[
            pl.BlockSpec((BLK, DIM), lambda rb, fc, be, bu: (rb * bu[rb], 0)),
            pl.BlockSpec((1, DIM, FFC),
                         lambda rb, fc, be, bu: (be[rb], 0, fc * bu[rb])),
            pl.BlockSpec((1, FFC, DIM),
                         lambda rb, fc, be, bu: (be[rb], fc * bu[rb], 0)),
        ],
        out_specs=pl.BlockSpec((BLK, DIM), lambda rb, fc, be, bu: (rb, 0)),
        scratch_shapes=[pltpu.VMEM((BLK, DIM), jnp.float32)],
    )
    return pl.pallas_call(
        _ffn_body,
        grid_spec=grid_spec,
        out_shape=jax.ShapeDtypeStruct((P, DIM), jnp.float32),
        compiler_params=pltpu.CompilerParams(
            dimension_semantics=("arbitrary", "arbitrary")),
    )(beo, beu, xp, W1, W2)


@functools.cache
def _combine_fn():
    return functools.partial(
        pl.kernel,
        out_type=jax.ShapeDtypeStruct((N, DIM), jnp.float32),
        mesh=_mesh(),
        scratch_types=[
            pltpu.VMEM((TPT,), jnp.int32),
            pltpu.VMEM((TPT,), jnp.int32),
            pltpu.VMEM((TPT,), jnp.float32),
            pltpu.VMEM((TPT,), jnp.float32),
            pltpu.VMEM((2, CH2, DIM), jnp.float32),
            pltpu.VMEM((2, CH2, DIM), jnp.float32),
            pltpu.VMEM((2, CH2, DIM), jnp.float32),
            pltpu.SemaphoreType.DMA,
            pltpu.SemaphoreType.DMA,
            pltpu.SemaphoreType.DMA,
            pltpu.SemaphoreType.DMA,
            pltpu.SemaphoreType.DMA,
            pltpu.SemaphoreType.DMA,
        ],
        compiler_params=pltpu.CompilerParams(needs_layout_passes=False),
    )(_combine_body)


def _combine_body(y_hbm, dest_hbm, w_hbm, out_hbm,
                  d0v, d1v, w0v, w1v, r0, r1, ob,
                  g00, g01, g10, g11, so0, so1):
    wid = lax.axis_index("s") * NC + lax.axis_index("c")
    t0 = wid * TPT
    pltpu.sync_copy(dest_hbm.at[pl.ds(t0, TPT)], d0v)
    pltpu.sync_copy(dest_hbm.at[pl.ds(N + t0, TPT)], d1v)
    pltpu.sync_copy(w_hbm.at[pl.ds(t0, TPT)], w0v)
    pltpu.sync_copy(w_hbm.at[pl.ds(N + t0, TPT)], w1v)

    nch = TPT // CH2
    g0s = (g00, g01)
    g1s = (g10, g11)
    sos = (so0, so1)

    def _gath(c, b):
        return (pltpu.async_copy(
                    y_hbm.at[d0v.at[pl.ds(c * CH2, CH2)]], r0.at[b], g0s[b]),
                pltpu.async_copy(
                    y_hbm.at[d1v.at[pl.ds(c * CH2, CH2)]], r1.at[b], g1s[b]))

    def _store(c, b):
        return pltpu.async_copy(
            ob.at[b], out_hbm.at[pl.ds(t0 + c * CH2, CH2)], sos[b])

    gats = {0: _gath(0, 0)}
    stores = {}
    for c in range(nch):
        b = c & 1
        ga, gb = gats[c]
        ga.wait()
        gb.wait()
        if c + 1 < nch:
            gats[c + 1] = _gath(c + 1, 1 - b)
        if c - 2 >= 0:
            stores[c - 2].wait()

        wv0 = w0v[pl.ds(c * CH2, CH2)]
        wv1 = w1v[pl.ds(c * CH2, CH2)]

        def cbody(t, _):
            tt = jnp.full((L,), t, jnp.int32)
            w0s = _gather16(wv0, tt)
            w1s = _gather16(wv1, tt)
            for kk in range(DIM // L):
                sl = pl.ds(kk * L, L)
                ob[b, t, sl] = w0s * r0[b, t, sl] + w1s * r1[b, t, sl]
            return 0

        lax.fori_loop(0, CH2, cbody, 0)
        stores[c] = _store(c, b)
    stores[nch - 2].wait()
    stores[nch - 1].wait()


def kernel(x, Wr, W1, W2):
    Bb, Tt, D = x.shape
    x_flat = x.reshape(N, D)
    e2, w2, cc = _router(x_flat, Wr)
    eflat = e2.reshape(NK)
    wflat = w2.reshape(NK)
    cnt = cc.reshape(NW, L)
    dest, xp, beo, beu = _dispatch_fn()(eflat, cnt, x_flat)
    y = _ffn(beo, beu, xp,
             W1.astype(jnp.bfloat16), W2.astype(jnp.bfloat16))
    out = _combine_fn()(y, dest, wflat)
    return out.reshape(Bb, Tt, D)


# speedup vs baseline: 1.9786x; 1.2344x over previous
import functools

import jax
import jax.numpy as jnp
from jax import lax
from jax.experimental import pallas as pl
from jax.experimental.pallas import tpu as pltpu
from jax.experimental.pallas import tpu_sc as plsc

DIM = 1024
FF = 4096
E = 8
K = 2
N = 4096
NK = N * K
NC, NS, L = 2, 16, 16
NW = NC * NS
S = NK // NW
TPT = N // NW
BLK = 512
BSH = 9
P = NK + E * BLK
RB = P // BLK
FFC = 2048
NFF = FF // FFC
RN = 1024
CH = 32
CH2 = 16

@functools.cache
def _mesh():
    return plsc.VectorSubcoreMesh(
        core_axis_name="c", subcore_axis_name="s",
        num_cores=NC, num_subcores=NS)


def _gather16(src, idx):
    return lax.gather(
        src,
        idx[:, None],
        lax.GatherDimensionNumbers(
            offset_dims=(), collapsed_slice_dims=(0,), start_index_map=(0,)),
        (1,),
        mode=lax.GatherScatterMode.PROMISE_IN_BOUNDS,
    )


def _router_body(x_ref, wr_ref, e2_ref, w2_ref, cc_ref):
    xb = x_ref[...]
    logits = jax.lax.dot_general(
        xb, wr_ref[...], (((1,), (1,)), ((), ())),
        preferred_element_type=jnp.float32)
    ids = lax.broadcasted_iota(jnp.int32, (RN, E), 1)
    m1 = jnp.max(logits, axis=1, keepdims=True)
    a1 = jnp.min(jnp.where(logits == m1, ids, E), axis=1)
    neg = jnp.finfo(jnp.float32).min
    l2 = jnp.where(ids == a1[:, None], neg, logits)
    m2 = jnp.max(l2, axis=1, keepdims=True)
    a2 = jnp.min(jnp.where(l2 == m2, ids, E), axis=1)
    g = 1.0 / (1.0 + jnp.exp(m2[:, 0] - m1[:, 0]))
    e2_ref[0] = a1
    e2_ref[1] = a2
    w2_ref[0] = g
    w2_ref[1] = 1.0 - g
    ids16 = lax.broadcasted_iota(jnp.int32, (RN, L), 1)
    grp = (lax.broadcasted_iota(jnp.int32, (RN // S, RN), 1) // S ==
           lax.broadcasted_iota(jnp.int32, (RN // S, RN), 0)).astype(jnp.float32)
    oh1 = (ids16 == a1[:, None]).astype(jnp.float32)
    oh2 = (ids16 == a2[:, None]).astype(jnp.float32)
    cc_ref[0] = jnp.dot(grp, oh1, preferred_element_type=jnp.float32
                        ).astype(jnp.int32).reshape(1, RN // S, L)
    cc_ref[1] = jnp.dot(grp, oh2, preferred_element_type=jnp.float32
                        ).astype(jnp.int32).reshape(1, RN // S, L)


def _router(x_flat, Wr):
    nblk = N // RN
    return pl.pallas_call(
        _router_body,
        grid=(nblk,),
        in_specs=[
            pl.BlockSpec((RN, DIM), lambda b: (b, 0)),
            pl.BlockSpec((E, DIM), lambda b: (0, 0)),
        ],
        out_specs=[
            pl.BlockSpec((2, RN), lambda b: (0, b)),
            pl.BlockSpec((2, RN), lambda b: (0, b)),
            pl.BlockSpec((2, 1, RN // S, L), lambda b: (0, b, 0, 0)),
        ],
        out_shape=[
            jax.ShapeDtypeStruct((2, N), jnp.int32),
            jax.ShapeDtypeStruct((2, N), jnp.float32),
            jax.ShapeDtypeStruct((2, nblk, RN // S, L), jnp.int32),
        ],
    )(x_flat, Wr)


@functools.cache
def _dispatch_fn():
    return functools.partial(
        pl.kernel,
        out_type=[
            jax.ShapeDtypeStruct((NK,), jnp.int32),
            jax.ShapeDtypeStruct((P, DIM), jnp.float32),
            jax.ShapeDtypeStruct((RB,), jnp.int32),
            jax.ShapeDtypeStruct((RB,), jnp.int32),
        ],
        mesh=_mesh(),
        scratch_types=[
            pltpu.VMEM((NW, L), jnp.int32),
            pltpu.VMEM((S,), jnp.int32),
            pltpu.VMEM((S // CH, CH), jnp.int32),
            pltpu.VMEM((S,), jnp.int32),
            pltpu.VMEM((2, CH, DIM), jnp.float32),
            pltpu.VMEM((-(-RB // L) * L,), jnp.int32),
            pltpu.VMEM((-(-RB // L) * L,), jnp.int32),
            pltpu.SemaphoreType.DMA,
            pltpu.SemaphoreType.DMA,
            pltpu.SemaphoreType.DMA,
            pltpu.SemaphoreType.DMA,
        ],
        compiler_params=pltpu.CompilerParams(needs_layout_passes=False),
    )(_dispatch_body)


def _dispatch_body(e_hbm, cnt_hbm, x_hbm,
                   dest_hbm, xp_hbm, beo_hbm, beu_hbm,
                   cnt_v, ev, destv, destf, xr2, beov, beuv,
                   sl0, sl1, ss0, ss1):
    wid = lax.axis_index("s") * NC + lax.axis_index("c")
    pltpu.sync_copy(cnt_hbm, cnt_v)
    pltpu.sync_copy(e_hbm.at[pl.ds(wid * S, S)], ev)

    lane = lax.broadcasted_iota(jnp.int32, (L,), 0)
    zero = jnp.zeros((L,), jnp.int32)
    tot = zero
    pre = zero
    for j in range(NW):
        row = cnt_v[j]
        tot = tot + row
        pre = pre + jnp.where(jnp.full((L,), j, jnp.int32) < wid, row, zero)
    padded = ((tot + (BLK - 1)) >> BSH) << BSH
    incl = plsc.cumsum(padded)
    po = incl - padded
    base = po + pre

    nv_per_row = CH // L

    def dbody(j, run):
        v = ev[pl.ds(j * L, L)]
        rank = zero
        hist = zero
        for e in range(E):
            m = v == e
            mi = m.astype(jnp.int32)
            inc = plsc.cumsum(mi)
            rank = jnp.where(m, inc - 1, rank)
            hist = jnp.where(lane == e, jnp.sum(mi), hist)
        dvec = _gather16(base + run, v) + rank
        destv[j // nv_per_row, pl.ds((j % nv_per_row) * L, L)] = dvec
        destf[pl.ds(j * L, L)] = dvec
        return run + hist

    lax.fori_loop(0, S // L, dbody, zero)

    pltpu.sync_copy(destf, dest_hbm.at[pl.ds(wid * S, S)])

    tb = jnp.where(wid < NW // 2, wid * S, wid * S - N)
    nch = S // CH
    sl = (sl0, sl1)
    ss = (ss0, ss1)

    def _load(c, b):
        return pltpu.async_copy(
            x_hbm.at[pl.ds(tb + c * CH, CH)], xr2.at[b], sl[b])

    def _scat(c, b):
        return pltpu.async_copy(xr2.at[b], xp_hbm.at[destv.at[c]], ss[b])

    loads = {0: _load(0, 0)}
    scats = {}
    for c in range(nch):
        b = c & 1
        loads[c].wait()
        if c + 1 < nch:
            if c - 1 >= 0:
                scats[c - 1].wait()
            loads[c + 1] = _load(c + 1, 1 - b)
        scats[c] = _scat(c, b)
    scats[nch - 2].wait()
    scats[nch - 1].wait()

    @pl.when(wid == 0)
    def _():
        for jb in range(-(-RB // L)):
            bstart = (lane + jb * L) * BLK
            acc = zero
            for e in range(E):
                th = _gather16(po, jnp.full((L,), e, jnp.int32))
                acc = acc + jnp.where(th <= bstart, 1, 0)
            bev = acc - 1
            beov[pl.ds(jb * L, L)] = bev
            rend = _gather16(po + tot, bev)
            beuv[pl.ds(jb * L, L)] = jnp.where(bstart < rend, 1, 0)
        pltpu.sync_copy(beov.at[pl.ds(0, RB)], beo_hbm)
        pltpu.sync_copy(beuv.at[pl.ds(0, RB)], beu_hbm)


def _ffn_body(be_ref, bu_ref, xp_ref, w1_ref, w2_ref, y_ref, acc):
    rb = pl.program_id(0)
    fc = pl.program_id(1)

    @pl.when(bu_ref[rb] == 1)
    def _():
        h = jnp.dot(xp_ref[...].astype(jnp.bfloat16),
                    w1_ref[0].astype(jnp.bfloat16),
                    preferred_element_type=jnp.float32)
        h = 0.5 * h * (1.0 + lax.erf(h * (2.0 ** -0.5)))
        p = jnp.dot(h.astype(jnp.bfloat16),
                    w2_ref[0].astype(jnp.bfloat16),
                    preferred_element_type=jnp.float32)

        @pl.when(fc == 0)
        def _():
            acc[...] = p

        @pl.when(fc > 0)
        def _():
            acc[...] += p

        @pl.when(fc == NFF - 1)
        def _():
            y_ref[...] = acc[...]


def _ffn(beo, beu, xp, W1, W2):
    grid_spec = pltpu.PrefetchScalarGridSpec(
        num_scalar_prefetch=2,
        grid=(RB, NFF),
        in_specs=[
            pl.BlockSpec((BLK, DIM), lambda rb, fc, be, bu: (rb * bu[rb], 0)),
            pl.BlockSpec((1, DIM, FFC),
                         lambda rb, fc, be, bu: (be[rb], 0, fc * bu[rb])),
            pl.BlockSpec((1, FFC, DIM),
                         lambda rb, fc, be, bu: (be[rb], fc * bu[rb], 0)),
        ],
        out_specs=pl.BlockSpec((BLK, DIM), lambda rb, fc, be, bu: (rb, 0)),
        scratch_shapes=[pltpu.VMEM((BLK, DIM), jnp.float32)],
    )
    return pl.pallas_call(
        _ffn_body,
        grid_spec=grid_spec,
        out_shape=jax.ShapeDtypeStruct((P, DIM), jnp.float32),
        compiler_params=pltpu.CompilerParams(
            dimension_semantics=("arbitrary", "arbitrary")),
    )(beo, beu, xp, W1, W2)


@functools.cache
def _combine_fn():
    return functools.partial(
        pl.kernel,
        out_type=jax.ShapeDtypeStruct((N, DIM), jnp.float32),
        mesh=_mesh(),
        scratch_types=[
            pltpu.VMEM((TPT,), jnp.int32),
            pltpu.VMEM((TPT,), jnp.int32),
            pltpu.VMEM((TPT,), jnp.float32),
            pltpu.VMEM((TPT,), jnp.float32),
            pltpu.VMEM((2, CH2, DIM), jnp.float32),
            pltpu.VMEM((2, CH2, DIM), jnp.float32),
            pltpu.VMEM((2, CH2, DIM), jnp.float32),
            pltpu.SemaphoreType.DMA,
            pltpu.SemaphoreType.DMA,
            pltpu.SemaphoreType.DMA,
            pltpu.SemaphoreType.DMA,
            pltpu.SemaphoreType.DMA,
            pltpu.SemaphoreType.DMA,
        ],
        compiler_params=pltpu.CompilerParams(needs_layout_passes=False),
    )(_combine_body)


def _combine_body(y_hbm, dest_hbm, w_hbm, out_hbm,
                  d0v, d1v, w0v, w1v, r0, r1, ob,
                  g00, g01, g10, g11, so0, so1):
    wid = lax.axis_index("s") * NC + lax.axis_index("c")
    t0 = wid * TPT
    pltpu.sync_copy(dest_hbm.at[pl.ds(t0, TPT)], d0v)
    pltpu.sync_copy(dest_hbm.at[pl.ds(N + t0, TPT)], d1v)
    pltpu.sync_copy(w_hbm.at[pl.ds(t0, TPT)], w0v)
    pltpu.sync_copy(w_hbm.at[pl.ds(N + t0, TPT)], w1v)

    nch = TPT // CH2
    g0s = (g00, g01)
    g1s = (g10, g11)
    sos = (so0, so1)

    def _gath(c, b):
        return (pltpu.async_copy(
                    y_hbm.at[d0v.at[pl.ds(c * CH2, CH2)]], r0.at[b], g0s[b]),
                pltpu.async_copy(
                    y_hbm.at[d1v.at[pl.ds(c * CH2, CH2)]], r1.at[b], g1s[b]))

    def _store(c, b):
        return pltpu.async_copy(
            ob.at[b], out_hbm.at[pl.ds(t0 + c * CH2, CH2)], sos[b])

    gats = {0: _gath(0, 0)}
    stores = {}
    for c in range(nch):
        b = c & 1
        ga, gb = gats[c]
        ga.wait()
        gb.wait()
        if c + 1 < nch:
            gats[c + 1] = _gath(c + 1, 1 - b)
        if c - 2 >= 0:
            stores[c - 2].wait()

        wv0 = w0v[pl.ds(c * CH2, CH2)]
        wv1 = w1v[pl.ds(c * CH2, CH2)]

        def cbody(t, _):
            tt = jnp.full((L,), t, jnp.int32)
            w0s = _gather16(wv0, tt)
            w1s = _gather16(wv1, tt)
            for kk in range(DIM // L):
                sl = pl.ds(kk * L, L)
                ob[b, t, sl] = w0s * r0[b, t, sl] + w1s * r1[b, t, sl]
            return 0

        lax.fori_loop(0, CH2, cbody, 0)
        stores[c] = _store(c, b)
    stores[nch - 2].wait()
    stores[nch - 1].wait()


def kernel(x, Wr, W1, W2):
    Bb, Tt, D = x.shape
    x_flat = x.reshape(N, D)
    e2, w2, cc = _router(x_flat, Wr)
    eflat = e2.reshape(NK)
    wflat = w2.reshape(NK)
    cnt = cc.reshape(NW, L)
    dest, xp, beo, beu = _dispatch_fn()(eflat, cnt, x_flat)
    y = _ffn(beo, beu, xp, W1, W2)
    out = _combine_fn()(y, dest, wflat)
    return out.reshape(Bb, Tt, D)
